# baseline (speedup 1.0000x reference)
import functools

import jax
import jax.numpy as jnp
from jax import lax
from jax.experimental import pallas as pl
from jax.experimental.pallas import tpu as pltpu
from jax.experimental.pallas import tpu_sc as plsc

SEQ = 200
DIM = 64
LANES = 16
VECS_PER_ROW = DIM // LANES
SCALE = 8.0

NUM_WORKERS = 32
BPW = 128
TILE_ROWS = 8
D_TILES = DIM // TILE_ROWS
NBUF = 4


def _embed_kernel(batch):
    assert batch == NUM_WORKERS * BPW
    n_chunks = SEQ
    mesh = plsc.VectorSubcoreMesh(core_axis_name="c", subcore_axis_name="s")

    @functools.partial(
        pl.kernel,
        mesh=mesh,
        out_type=jax.ShapeDtypeStruct((SEQ * D_TILES * NUM_WORKERS, 1024),
                                      jnp.float32),
        scratch_types=[
            pltpu.VMEM((SEQ, BPW), jnp.int32),
            pltpu.VMEM((SEQ, DIM), jnp.float32),
            pltpu.VMEM((BPW, DIM), jnp.float32),
            pltpu.VMEM((BPW, DIM), jnp.float32),
            pltpu.VMEM((BPW, DIM), jnp.float32),
            pltpu.VMEM((BPW, DIM), jnp.float32),
            pltpu.VMEM((DIM * BPW,), jnp.float32),
            pltpu.VMEM((DIM * BPW,), jnp.float32),
            pltpu.VMEM((DIM * BPW,), jnp.float32),
            pltpu.VMEM((DIM * BPW,), jnp.float32),
            pltpu.SemaphoreType.DMA,
            pltpu.SemaphoreType.DMA,
            pltpu.SemaphoreType.DMA,
            pltpu.SemaphoreType.DMA,
            pltpu.SemaphoreType.DMA,
            pltpu.SemaphoreType.DMA,
            pltpu.SemaphoreType.DMA,
            pltpu.SemaphoreType.DMA,
        ],
        compiler_params=pltpu.CompilerParams(use_tc_tiling_on_sc=False, needs_layout_passes=False),
    )
    def body(idx_hbm, table_hbm, pos_hbm, out_hbm,
             idx_v, pos_v, g0, g1, g2, g3, t0, t1, t2, t3,
             sg0, sg1, sg2, sg3, ss0, ss1, ss2, ss3):
        gbufs = (g0, g1, g2, g3)
        tbufs = (t0, t1, t2, t3)
        sgs = (sg0, sg1, sg2, sg3)
        sss = (ss0, ss1, ss2, ss3)
        wid = lax.axis_index("s") * 2 + lax.axis_index("c")

        pltpu.sync_copy(pos_hbm, pos_v)
        pltpu.sync_copy(idx_hbm.at[wid], idx_v)

        iota128 = lax.iota(jnp.int32, LANES) * 128

        def start_gather(s, b):
            pltpu.async_copy(table_hbm.at[idx_v.at[s]], gbufs[b], sgs[b])

        def wait_gather(b):
            pltpu.make_async_copy(
                table_hbm.at[pl.ds(0, BPW)], gbufs[b], sgs[b]
            ).wait()

        def start_scatter(s, b):
            for td in range(D_TILES):
                pltpu.async_copy(
                    tbufs[b].at[pl.ds(td * 1024, 1024)],
                    out_hbm.at[(s * D_TILES + td) * NUM_WORKERS + wid],
                    sss[b],
                )

        def wait_scatter(b):
            for td in range(D_TILES):
                pltpu.make_async_copy(
                    out_hbm.at[td],
                    tbufs[b].at[pl.ds(td * 1024, 1024)],
                    sss[b],
                ).wait()

        def compute(s, b):
            g = gbufs[b]
            t = tbufs[b]
            p = [pos_v[s, pl.ds(q * LANES, LANES)] for q in range(VECS_PER_ROW)]

            def bb_body(bb, c2):
                for q in range(VECS_PER_ROW):
                    v = g[bb, pl.ds(q * LANES, LANES)]
                    tgt = iota128 + (q * LANES * 128 + bb)
                    plsc.store_scatter(t, [tgt], v * SCALE + p[q])
                return c2

            lax.fori_loop(0, BPW, bb_body, 0, unroll=2)

        start_gather(0, 0)
        start_gather(1, 1)
        start_gather(2, 2)

        def outer(k, carry):
            for b in range(NBUF):
                s = NBUF * k + b
                wait_gather(b)
                compute(s, b)
                start_scatter(s, b)
                b2 = (b + 3) % NBUF

                @pl.when(s <= n_chunks - NBUF)
                def _():
                    @pl.when(s >= 1)
                    def _():
                        wait_scatter(b2)

                    start_gather(s + NBUF - 1, b2)

            return carry

        lax.fori_loop(0, n_chunks // NBUF, outer, 0)

        wait_scatter(0)
        wait_scatter(1)
        wait_scatter(2)
        wait_scatter(3)

    return body


def kernel(inputs, token_table, pos_table):
    batch, seq = inputs.shape
    idxp = (inputs.astype(jnp.int32)
            .reshape(NUM_WORKERS, BPW, seq)
            .transpose(0, 2, 1))
    out2 = _embed_kernel(batch)(idxp, token_table, pos_table)
    x = out2.reshape(seq, D_TILES, NUM_WORKERS, TILE_ROWS, BPW)
    return (x.transpose(2, 4, 0, 1, 3)
            .reshape(batch, seq, DIM))


# speedup vs baseline: 1.5764x; 1.5764x over previous
import functools

import jax
import jax.numpy as jnp
from jax import lax
from jax.experimental import pallas as pl
from jax.experimental.pallas import tpu as pltpu
from jax.experimental.pallas import tpu_sc as plsc

SEQ = 200
DIM = 64
LANES = 16
VECS_PER_ROW = DIM // LANES
SCALE = 8.0

NUM_WORKERS = 32
IDX_MINOR = 100
CH_SEQ = 2
CH_ROWS = CH_SEQ * SEQ
G_PER_CHUNK = CH_ROWS // IDX_MINOR
NBUF = 3


def _embed_kernel(rows_total):
    rows_per_w = rows_total // NUM_WORKERS
    n_chunks = rows_per_w // CH_ROWS
    ring_chunks = n_chunks - 1
    assert ring_chunks % NBUF == 0
    mesh = plsc.VectorSubcoreMesh(core_axis_name="c", subcore_axis_name="s")

    @functools.partial(
        pl.kernel,
        mesh=mesh,
        out_type=jax.ShapeDtypeStruct((rows_total, DIM), jnp.float32),
        scratch_types=[
            pltpu.VMEM((rows_per_w // IDX_MINOR, IDX_MINOR), jnp.int32),
            pltpu.VMEM((SEQ, DIM), jnp.float32),
            pltpu.VMEM((CH_ROWS, DIM), jnp.float32),
            pltpu.VMEM((CH_ROWS, DIM), jnp.float32),
            pltpu.VMEM((CH_ROWS, DIM), jnp.float32),
            pltpu.SemaphoreType.DMA,
            pltpu.SemaphoreType.DMA,
            pltpu.SemaphoreType.DMA,
            pltpu.SemaphoreType.DMA,
            pltpu.SemaphoreType.DMA,
            pltpu.SemaphoreType.DMA,
        ],
        compiler_params=pltpu.CompilerParams(use_tc_tiling_on_sc=False),
    )
    def body(idx_hbm, table_hbm, pos_hbm, out_hbm,
             idx_v, pos_v, buf0, buf1, buf2, sg0, sg1, sg2, ss0, ss1, ss2):
        bufs = (buf0, buf1, buf2)
        sgs = (sg0, sg1, sg2)
        sss = (ss0, ss1, ss2)
        wid = lax.axis_index("s") * 2 + lax.axis_index("c")
        row_base = wid * rows_per_w

        pltpu.sync_copy(pos_hbm, pos_v)
        pltpu.sync_copy(idx_hbm.at[wid], idx_v)

        def start_gather(g, b):
            for j in range(G_PER_CHUNK):
                pltpu.async_copy(
                    table_hbm.at[idx_v.at[g * G_PER_CHUNK + j]],
                    bufs[b].at[pl.ds(j * IDX_MINOR, IDX_MINOR)],
                    sgs[b],
                )

        def wait_gather(b):
            pltpu.make_async_copy(
                table_hbm.at[pl.ds(0, CH_ROWS)], bufs[b], sgs[b]
            ).wait()

        def start_scatter(g, b):
            pltpu.async_copy(
                bufs[b],
                out_hbm.at[pl.ds(row_base + g * CH_ROWS, CH_ROWS)],
                sss[b],
            )

        def wait_scatter(b):
            pltpu.make_async_copy(
                table_hbm.at[pl.ds(0, CH_ROWS)], bufs[b], sss[b]
            ).wait()

        def compute(b):
            buf = bufs[b]

            def row_body(rr, c2):
                for q in range(VECS_PER_ROW):
                    p = pos_v[rr, pl.ds(q * LANES, LANES)]
                    for rep in range(CH_SEQ):
                        sl = (rep * SEQ + rr, pl.ds(q * LANES, LANES))
                        buf[sl] = buf[sl] * SCALE + p
                return c2

            lax.fori_loop(0, SEQ, row_body, 0, unroll=2)

        start_gather(0, 0)
        start_gather(1, 1)

        def outer(k, carry):
            for b in range(NBUF):
                g = NBUF * k + b
                wait_gather(b)
                compute(b)
                start_scatter(g, b)
                b2 = (b + 2) % NBUF

                @pl.when(g <= ring_chunks - 3)
                def _():
                    @pl.when(g >= 1)
                    def _():
                        wait_scatter(b2)

                    start_gather(g + 2, b2)

            return carry

        lax.fori_loop(0, ring_chunks // NBUF, outer, 0)

        wait_scatter(0)
        start_gather(n_chunks - 1, 0)
        wait_gather(0)
        compute(0)
        start_scatter(n_chunks - 1, 0)
        wait_scatter(0)
        wait_scatter(1)
        wait_scatter(2)

    return body


def kernel(inputs, token_table, pos_table):
    batch, seq = inputs.shape
    rows_total = batch * seq
    idx3 = inputs.reshape(
        NUM_WORKERS, rows_total // (NUM_WORKERS * IDX_MINOR), IDX_MINOR
    ).astype(jnp.int32)
    out = _embed_kernel(rows_total)(idx3, token_table, pos_table)
    return out.reshape(batch, seq, DIM)
